# Initial kernel scaffold; baseline (speedup 1.0000x reference)
import jax
import jax.numpy as jnp
from jax.experimental import pallas as pl


def kernel(points, uncertainty):
    raise NotImplementedError("write your pallas kernel here")


# speedup vs baseline: 94.7369x; 94.7369x over previous
import functools

import jax
import jax.numpy as jnp
from jax import lax
from jax.experimental import pallas as pl
from jax.experimental.pallas import tpu as pltpu
from jax.experimental.pallas import tpu_sc as plsc

_NC = 2
_NS = 16
_NW = _NC * _NS

_CHUNK = 2048


def _elu_plus_one_body(x_ref, o_ref):
    x = x_ref[...]
    o_ref[...] = jnp.where(x > 0, x + 1.0, jnp.exp(x))


def _elu_plus_one_tc(table_flat):
    n = table_flat.shape[0]
    x2d = table_flat.reshape(1000, n // 1000)
    out = pl.pallas_call(
        _elu_plus_one_body,
        out_shape=jax.ShapeDtypeStruct(x2d.shape, jnp.float32),
    )(x2d)
    return out.reshape(n)


def _make_sc_gather(n_idx):
    per_tile = n_idx // _NW
    n_chunks = per_tile // _CHUNK
    mesh = plsc.VectorSubcoreMesh(core_axis_name="c", subcore_axis_name="s")

    @functools.partial(
        pl.kernel,
        mesh=mesh,
        out_type=jax.ShapeDtypeStruct((n_idx,), jnp.float32),
        scratch_types=[
            pltpu.VMEM((_CHUNK,), jnp.int32),
            pltpu.VMEM((_CHUNK,), jnp.float32),
            pltpu.SemaphoreType.DMA,
        ],
    )
    def gather_kernel(table_hbm, idx_hbm, out_hbm, idx_v, rows_v, sem):
        wid = lax.axis_index("s") * _NC + lax.axis_index("c")
        base = wid * per_tile

        def body(i, _):
            off = base + i * _CHUNK
            pltpu.sync_copy(idx_hbm.at[pl.ds(off, _CHUNK)], idx_v)
            pltpu.async_copy(table_hbm.at[idx_v], rows_v, sem).wait()
            pltpu.sync_copy(rows_v, out_hbm.at[pl.ds(off, _CHUNK)])
            return ()

        lax.fori_loop(0, n_chunks, body, ())

    return gather_kernel


def kernel(points, uncertainty):
    b, t = points.shape
    table = _elu_plus_one_tc(uncertainty.reshape(-1))
    idx = points.reshape(-1)
    out = _make_sc_gather(idx.shape[0])(table, idx)
    return out.reshape(b, t, 1)


# speedup vs baseline: 118.2827x; 1.2485x over previous
import functools

import jax
import jax.numpy as jnp
from jax import lax
from jax.experimental import pallas as pl
from jax.experimental.pallas import tpu as pltpu
from jax.experimental.pallas import tpu_sc as plsc

_NC = 2
_NS = 16
_NW = _NC * _NS

_CHUNK_ROWS = 64
_NBUF = 2


def _elu_plus_one_body(x_ref, o_ref):
    x = x_ref[...]
    o_ref[...] = jnp.where(x > 0, x + 1.0, jnp.exp(x))


def _elu_plus_one_tc(table_flat):
    n = table_flat.shape[0]
    x2d = table_flat.reshape(1000, n // 1000)
    out = pl.pallas_call(
        _elu_plus_one_body,
        out_shape=jax.ShapeDtypeStruct(x2d.shape, jnp.float32),
    )(x2d)
    return out.reshape(n)


def _make_sc_gather(n_idx):
    per_tile = n_idx // _NW
    chunk = _CHUNK_ROWS * 200
    n_chunks = per_tile // chunk
    mesh = plsc.VectorSubcoreMesh(core_axis_name="c", subcore_axis_name="s")

    @functools.partial(
        pl.kernel,
        mesh=mesh,
        out_type=jax.ShapeDtypeStruct((n_idx,), jnp.float32),
        scratch_types=[pltpu.VMEM((chunk,), jnp.int32)] * _NBUF
        + [pltpu.VMEM((chunk,), jnp.float32)] * _NBUF
        + [pltpu.SemaphoreType.DMA] * (3 * _NBUF),
    )
    def gather_kernel(table_hbm, idx_hbm, out_hbm, *scratch):
        idx_v = scratch[0:_NBUF]
        rows_v = scratch[_NBUF : 2 * _NBUF]
        sems = scratch[2 * _NBUF :]
        sem_i = sems[0:_NBUF]
        sem_g = sems[_NBUF : 2 * _NBUF]
        sem_o = sems[2 * _NBUF :]
        wid = lax.axis_index("s") * _NC + lax.axis_index("c")
        base = wid * per_tile

        def idx_start(i):
            b = i % _NBUF
            src = idx_hbm.at[pl.ds(base + i * chunk, chunk)]
            return pltpu.async_copy(src, idx_v[b], sem_i[b])

        def gather_start(i):
            b = i % _NBUF
            return pltpu.async_copy(
                table_hbm.at[idx_v[b]], rows_v[b], sem_g[b]
            )

        def out_start(i):
            b = i % _NBUF
            dst = out_hbm.at[pl.ds(base + i * chunk, chunk)]
            return pltpu.async_copy(rows_v[b], dst, sem_o[b])

        cp_i = {0: idx_start(0)}
        g = {}
        o = {}
        for i in range(n_chunks):
            cp_i[i].wait()
            if i >= _NBUF:
                o[i - _NBUF].wait()
            g[i] = gather_start(i)
            if i >= 1:
                g[i - 1].wait()
                o[i - 1] = out_start(i - 1)
            if i + 1 < n_chunks:
                cp_i[i + 1] = idx_start(i + 1)
        last = n_chunks - 1
        g[last].wait()
        o[last] = out_start(last)
        for j in range(max(0, n_chunks - _NBUF), n_chunks):
            o[j].wait()

    return gather_kernel


def kernel(points, uncertainty):
    b, t = points.shape
    table = _elu_plus_one_tc(uncertainty.reshape(-1))
    idx = points.reshape(-1)
    out = _make_sc_gather(idx.shape[0])(table, idx)
    return out.reshape(b, t, 1)


# speedup vs baseline: 149.9546x; 1.2678x over previous
import functools

import jax
import jax.numpy as jnp
from jax import lax
from jax.experimental import pallas as pl
from jax.experimental.pallas import tpu as pltpu
from jax.experimental.pallas import tpu_sc as plsc

_NC = 2
_NS = 16
_NW = _NC * _NS

_CHUNK = 12800
_NBUF = 2


def _make_sc_gather(n_tab, n_idx):
    per_tile = n_idx // _NW
    n_chunks = per_tile // _CHUNK
    mesh = plsc.VectorSubcoreMesh(core_axis_name="c", subcore_axis_name="s")

    @functools.partial(
        pl.kernel,
        mesh=mesh,
        out_type=jax.ShapeDtypeStruct((n_idx,), jnp.float32),
        scratch_types=[pltpu.VMEM_SHARED((n_tab,), jnp.float32)]
        + [pltpu.VMEM((_CHUNK,), jnp.int32)] * _NBUF
        + [pltpu.VMEM((_CHUNK,), jnp.float32)] * _NBUF
        + [pltpu.SemaphoreType.DMA] * (3 * _NBUF),
    )
    def gather_kernel(table_hbm, idx_hbm, out_hbm, spm, *rest):
        idx_v = rest[0:_NBUF]
        rows_v = rest[_NBUF : 2 * _NBUF]
        sems = rest[2 * _NBUF :]
        sem_i = sems[0:_NBUF]
        sem_g = sems[_NBUF : 2 * _NBUF]
        sem_o = sems[2 * _NBUF :]
        s = lax.axis_index("s")
        wid = s * _NC + lax.axis_index("c")

        @pl.when(s == 0)
        def _stage():
            pltpu.sync_copy(table_hbm, spm)

        plsc.subcore_barrier()
        base = wid * per_tile

        def idx_start(i):
            b = i % _NBUF
            src = idx_hbm.at[pl.ds(base + i * _CHUNK, _CHUNK)]
            return pltpu.async_copy(src, idx_v[b], sem_i[b])

        def gather_start(i):
            b = i % _NBUF
            return pltpu.async_copy(spm.at[idx_v[b]], rows_v[b], sem_g[b])

        def out_start(i):
            b = i % _NBUF
            dst = out_hbm.at[pl.ds(base + i * _CHUNK, _CHUNK)]
            return pltpu.async_copy(rows_v[b], dst, sem_o[b])

        def elu_rows(b):
            def body(j, _):
                v = rows_v[b][pl.ds(j * 16, 16)]
                rows_v[b][pl.ds(j * 16, 16)] = jnp.where(v > 0, v + 1.0, jnp.exp(v))
                return ()

            lax.fori_loop(0, _CHUNK // 16, body, ())

        cp = {0: idx_start(0)}
        g = {}
        o = {}
        for i in range(n_chunks):
            cp[i].wait()
            if i >= _NBUF:
                o[i - _NBUF].wait()
            g[i] = gather_start(i)
            if i >= 1:
                g[i - 1].wait()
                if i + 1 < n_chunks:
                    cp[i + 1] = idx_start(i + 1)
                elu_rows((i - 1) % _NBUF)
                o[i - 1] = out_start(i - 1)
            elif i + 1 < n_chunks:
                cp[i + 1] = idx_start(i + 1)
        g[n_chunks - 1].wait()
        elu_rows((n_chunks - 1) % _NBUF)
        o[n_chunks - 1] = out_start(n_chunks - 1)
        for j in range(max(0, n_chunks - _NBUF), n_chunks):
            o[j].wait()

    return gather_kernel


def kernel(points, uncertainty):
    b, t = points.shape
    table = uncertainty.reshape(-1)
    idx = points.reshape(-1)
    out = _make_sc_gather(table.shape[0], idx.shape[0])(table, idx)
    return out.reshape(b, t, 1)
